# baseline (speedup 1.0000x reference)
import functools

import jax
import jax.numpy as jnp
from jax import lax
from jax.experimental import pallas as pl
from jax.experimental.pallas import tpu as pltpu
from jax.experimental.pallas import tpu_sc as plsc

M = 18432
N_CODES = 8192
DIM = 256
BETA = 0.25

BM = 1024
BN = 2048
N_CHUNKS = N_CODES // BN
GPC = BN // 128


def _argmin_body(xn_ref, en_ref, xm2_ref, ebf_ref, idx_ref, dsum_ref):
    xm2 = xm2_ref[...]
    xn = xn_ref[...]

    best = jnp.full((BM, 128), jnp.inf, dtype=jnp.float32)
    bbase = jnp.zeros((BM, 128), dtype=jnp.int32)
    for c in range(N_CHUNKS):
        ebf = ebf_ref[pl.ds(c * BN, BN), :]
        nmm2 = lax.dot_general(
            xm2, ebf, (((1,), (1,)), ((), ())),
            preferred_element_type=jnp.float32)
        for g in range(GPC):
            en_g = en_ref[pl.ds(c * BN + g * 128, 128)]
            dg = (xn + en_g[None, :]) + nmm2[:, g * 128:(g + 1) * 128]
            m = dg < best
            best = jnp.where(m, dg, best)
            bbase = jnp.where(m, c * BN + g * 128, bbase)

    lane = lax.broadcasted_iota(jnp.int32, (BM, 128), 1)
    fbest = jnp.min(best, axis=1)
    bidx = jnp.min(jnp.where(best == fbest[:, None], bbase + lane, N_CODES),
                   axis=1)

    idx_ref[...] = bidx

    @pl.when(pl.program_id(0) == 0)
    def _():
        dsum_ref[...] = jnp.zeros((1, 1), jnp.float32)

    dsum_ref[...] += jnp.sum(fbest).reshape(1, 1)


def _tc_argmin(xn, en, xbf, ebf):
    rows = xbf.shape[0]
    grid = (rows // BM,)
    return pl.pallas_call(
        _argmin_body,
        grid=grid,
        in_specs=[
            pl.BlockSpec((BM, 1), lambda i: (i, 0)),
            pl.BlockSpec((N_CODES,), lambda i: (0,)),
            pl.BlockSpec((BM, DIM), lambda i: (i, 0)),
            pl.BlockSpec((N_CODES, DIM), lambda i: (0, 0)),
        ],
        out_specs=[
            pl.BlockSpec((BM,), lambda i: (i,)),
            pl.BlockSpec((1, 1), lambda i: (0, 0)),
        ],
        out_shape=[
            jax.ShapeDtypeStruct((rows,), jnp.int32),
            jax.ShapeDtypeStruct((1, 1), jnp.float32),
        ],
    )(xn, en, xbf, ebf)


_NW = 32


@functools.cache
def _sc_gather_kernel(rows):
    bpw = rows // _NW
    ch = min(bpw, 288)

    @functools.partial(
        pl.kernel,
        out_type=jax.ShapeDtypeStruct((rows, DIM), jnp.float32),
        mesh=plsc.VectorSubcoreMesh(core_axis_name="c", subcore_axis_name="s"),
        scratch_types=[
            pltpu.VMEM((bpw,), jnp.int32),
            pltpu.VMEM((ch, DIM), jnp.float32),
            pltpu.SemaphoreType.DMA,
        ],
    )
    def _sc_gather(emb_hbm, idx_hbm, out_hbm, idx_v, rows_v, sem):
        wid = lax.axis_index("s") * 2 + lax.axis_index("c")
        base = wid * bpw
        pltpu.sync_copy(idx_hbm.at[pl.ds(base, bpw)], idx_v)
        for c in range(bpw // ch):
            pltpu.async_copy(
                emb_hbm.at[idx_v.at[pl.ds(c * ch, ch)]], rows_v, sem).wait()
            pltpu.sync_copy(rows_v, out_hbm.at[pl.ds(base + c * ch, ch)])

    return _sc_gather


def kernel(x, embeddings):
    xf = x.reshape(-1, DIM)
    xn = jnp.sum(xf ** 2, axis=1, keepdims=True)
    en = jnp.sum(embeddings ** 2, axis=1)
    xm2 = (xf * -2.0).astype(jnp.bfloat16)
    ebf = embeddings.astype(jnp.bfloat16)
    h = M // 2
    idx0, dsum0 = _tc_argmin(xn[:h], en, xm2[:h], ebf)
    q0 = _sc_gather_kernel(h)(embeddings, idx0)
    idx1, dsum1 = _tc_argmin(xn[h:], en, xm2[h:], ebf)
    q1 = _sc_gather_kernel(h)(embeddings, idx1)
    quantized = jnp.concatenate([q0, q1], axis=0)
    loss = (BETA / (M * DIM)) * (dsum0[0, 0] + dsum1[0, 0])
    return quantized.reshape(x.shape), loss


# speedup vs baseline: 1.0462x; 1.0462x over previous
import functools

import jax
import jax.numpy as jnp
from jax import lax
from jax.experimental import pallas as pl
from jax.experimental.pallas import tpu as pltpu
from jax.experimental.pallas import tpu_sc as plsc

M = 18432
N_CODES = 8192
DIM = 256
BETA = 0.25

BM = 1024
BN = 2048
N_CHUNKS = N_CODES // BN
GPC = BN // 128


def _argmin_body(xn_ref, en_ref, xm2_ref, ebf_ref, idx_ref, dsum_ref):
    xm2 = xm2_ref[...]
    xn = xn_ref[...]

    best = jnp.full((BM, 128), jnp.inf, dtype=jnp.float32)
    bjf = jnp.zeros((BM, 128), dtype=jnp.float32)
    for c in range(N_CHUNKS):
        ebf = ebf_ref[pl.ds(c * BN, BN), :]
        nmm2 = lax.dot_general(
            xm2, ebf, (((1,), (1,)), ((), ())),
            preferred_element_type=jnp.float32)
        for g in range(GPC):
            en_g = en_ref[pl.ds(c * BN + g * 128, 128)]
            dg = (xn + en_g[None, :]) + nmm2[:, g * 128:(g + 1) * 128]
            m = dg < best
            best = jnp.where(m, dg, best)
            bjf = jnp.where(m, float(c * BN + g * 128), bjf)

    lanef = lax.broadcasted_iota(jnp.int32, (BM, 128), 1).astype(jnp.float32)
    fbest = jnp.min(best, axis=1)
    bidx = jnp.min(jnp.where(best == fbest[:, None], bjf + lanef, jnp.inf),
                   axis=1).astype(jnp.int32)

    idx_ref[...] = bidx

    @pl.when(pl.program_id(0) == 0)
    def _():
        dsum_ref[...] = jnp.zeros((1, 1), jnp.float32)

    dsum_ref[...] += jnp.sum(fbest).reshape(1, 1)


def _tc_argmin(xn, en, xbf, ebf):
    grid = (M // BM,)
    return pl.pallas_call(
        _argmin_body,
        grid=grid,
        in_specs=[
            pl.BlockSpec((BM, 1), lambda i: (i, 0)),
            pl.BlockSpec((N_CODES,), lambda i: (0,)),
            pl.BlockSpec((BM, DIM), lambda i: (i, 0)),
            pl.BlockSpec((N_CODES, DIM), lambda i: (0, 0)),
        ],
        out_specs=[
            pl.BlockSpec((BM,), lambda i: (i,)),
            pl.BlockSpec((1, 1), lambda i: (0, 0)),
        ],
        out_shape=[
            jax.ShapeDtypeStruct((M,), jnp.int32),
            jax.ShapeDtypeStruct((1, 1), jnp.float32),
        ],
    )(xn, en, xbf, ebf)


_NW = 32
_BPW = M // _NW
_CH = 288


@functools.cache
def _sc_gather_kernel():
    @functools.partial(
        pl.kernel,
        out_type=jax.ShapeDtypeStruct((M, DIM), jnp.float32),
        mesh=plsc.VectorSubcoreMesh(core_axis_name="c", subcore_axis_name="s"),
        scratch_types=[
            pltpu.VMEM((_BPW,), jnp.int32),
            pltpu.VMEM((_CH, DIM), jnp.float32),
            pltpu.SemaphoreType.DMA,
        ],
    )
    def _sc_gather(emb_hbm, idx_hbm, out_hbm, idx_v, rows_v, sem):
        wid = lax.axis_index("s") * 2 + lax.axis_index("c")
        base = wid * _BPW
        pltpu.sync_copy(idx_hbm.at[pl.ds(base, _BPW)], idx_v)
        for c in range(_BPW // _CH):
            pltpu.async_copy(
                emb_hbm.at[idx_v.at[pl.ds(c * _CH, _CH)]], rows_v, sem).wait()
            pltpu.sync_copy(rows_v, out_hbm.at[pl.ds(base + c * _CH, _CH)])

    return _sc_gather


def kernel(x, embeddings):
    xf = x.reshape(-1, DIM)
    xn = jnp.sum(xf ** 2, axis=1, keepdims=True)
    en = jnp.sum(embeddings ** 2, axis=1)
    xm2 = (xf * -2.0).astype(jnp.bfloat16)
    ebf = embeddings.astype(jnp.bfloat16)
    idx, dsum = _tc_argmin(xn, en, xm2, ebf)
    quantized = _sc_gather_kernel()(embeddings, idx)
    loss = (BETA / (M * DIM)) * dsum[0, 0]
    return quantized.reshape(x.shape), loss
